# Initial kernel scaffold; baseline (speedup 1.0000x reference)
import jax
import jax.numpy as jnp
from jax.experimental import pallas as pl


def kernel(preds_0, preds_1, preds_2, targets_box, targets_cls):
    raise NotImplementedError("write your pallas kernel here")


# speedup vs baseline: 1.3861x; 1.3861x over previous
import functools

import jax
import jax.numpy as jnp
from jax import lax
from jax.experimental import pallas as pl
from jax.experimental.pallas import tpu as pltpu
from jax.experimental.pallas import tpu_sc as plsc

_BS = 16
_NOBJ = 32
_C = 144
_NCLS = 80
_NSLOT = 84
_LANES = 16

_LEVELS = ((80, 80), (40, 40), (20, 20))


def _sigmoid(x):
    return 1.0 / (1.0 + jnp.exp(-x))


def _softplus_neg(a):
    z = jnp.exp(-a)
    u = z / (z + 2.0)
    u2 = u * u
    return 2.0 * u * (1.0 + u2 * (1.0 / 3.0 + u2 * (1.0 / 5.0
                      + u2 * (1.0 / 7.0 + u2 * (1.0 / 9.0)))))


def _sc_body(p0, p1, p2, tb, tcls, out, d0_v, d1_v, d2_v, d3_v, idx_v,
             vals_v, tbx_v, tby_v, tbw_v, tbh_v, cls_v, cells_v, res_v, sem):
    wid = lax.axis_index("s") * 2 + lax.axis_index("c")
    b = wid // 2
    oh = wid % 2
    iota = lax.iota(jnp.int32, _LANES)

    off_b = b * _NOBJ
    off_my = off_b + oh * _LANES
    pltpu.sync_copy(tb.at[pl.ds(off_b, _NOBJ)], tbx_v)
    pltpu.sync_copy(tb.at[pl.ds(512 + off_b, _NOBJ)], tby_v)
    pltpu.sync_copy(tb.at[pl.ds(1024 + off_my, _LANES)], tbw_v)
    pltpu.sync_copy(tb.at[pl.ds(1536 + off_my, _LANES)], tbh_v)
    pltpu.sync_copy(tcls.at[pl.ds(off_my, _LANES)], cls_v)

    x_lo = tbx_v[pl.ds(0, _LANES)]
    x_hi = tbx_v[pl.ds(_LANES, _LANES)]
    y_lo = tby_v[pl.ds(0, _LANES)]
    y_hi = tby_v[pl.ds(_LANES, _LANES)]
    bw = tbw_v[...]
    bh = tbh_v[...]
    cls_vec = cls_v[...]
    ohv = jnp.broadcast_to(oh, (_LANES,))
    my_rank = iota + oh * _LANES

    acc_box = jnp.zeros((_LANES,), jnp.float32)
    acc_cls = jnp.zeros((_LANES,), jnp.float32)
    acc_obj = jnp.zeros((_LANES,), jnp.float32)

    tables = (p0, p1, p2)
    for lvl, (h, w) in enumerate(_LEVELS):
        hw = h * w
        inv_n = 1.0 / float(_BS * hw)

        if lvl < 2:
            n_el = hw // 2
            el_off = oh * n_el
            scale = inv_n
        else:
            n_el = hw
            el_off = 0
            scale = inv_n * 0.5
        dbufs = (d0_v, d1_v, d2_v, d3_v)
        for c in range(4):
            e0 = (b * _C + c) * hw + el_off
            pltpu.sync_copy(tables[lvl].at[pl.ds(e0, n_el)],
                            dbufs[c].at[pl.ds(0, n_el)])

        def dense_step(i, acc):
            base = pl.multiple_of(i * _LANES, _LANES)
            v = (d0_v[pl.ds(base, _LANES)]
                 + d1_v[pl.ds(base, _LANES)]
                 + d2_v[pl.ds(base, _LANES)]
                 + d3_v[pl.ds(base, _LANES)]) * 0.25
            s = _sigmoid(v)
            return acc + s * s * scale

        acc_obj = lax.fori_loop(0, n_el // _LANES, dense_step, acc_obj)

        cx_lo = x_lo * float(w)
        cy_lo = y_lo * float(h)
        gx_lo = jnp.clip(cx_lo.astype(jnp.int32), 0, w - 1)
        gy_lo = jnp.clip(cy_lo.astype(jnp.int32), 0, h - 1)
        cx_hi = x_hi * float(w)
        cy_hi = y_hi * float(h)
        gx_hi = jnp.clip(cx_hi.astype(jnp.int32), 0, w - 1)
        gy_hi = jnp.clip(cy_hi.astype(jnp.int32), 0, h - 1)
        cells_v[pl.ds(0, _LANES)] = gy_lo * w + gx_lo
        cells_v[pl.ds(_LANES, _LANES)] = gy_hi * w + gx_hi

        is_lo = ohv == 0
        my_cx = jnp.where(is_lo, cx_lo, cx_hi)
        my_cy = jnp.where(is_lo, cy_lo, cy_hi)
        my_gx = jnp.where(is_lo, gx_lo, gx_hi)
        my_gy = jnp.where(is_lo, gy_lo, gy_hi)
        fx = my_cx - my_gx.astype(jnp.float32)
        fy = my_cy - my_gy.astype(jnp.float32)
        base_sp = my_gy * w + my_gx
        my_cell = base_sp

        bbase = b * _C * hw

        def idx_step(s, carry):
            c = jnp.where(s < 4, s, s + 60)
            off = pl.multiple_of(s * _LANES, _LANES)
            idx_v[pl.ds(off, _LANES)] = base_sp + (bbase + c * hw)
            return carry

        lax.fori_loop(0, _NSLOT, idx_step, 0)
        for j in range(12):
            o = j * 112
            pltpu.async_copy(tables[lvl].at[idx_v.at[pl.ds(o, 112)]],
                             vals_v.at[pl.ds(o, 112)], sem).wait()

        v0 = vals_v[pl.ds(0, _LANES)]
        v1 = vals_v[pl.ds(_LANES, _LANES)]
        v2 = vals_v[pl.ds(2 * _LANES, _LANES)]
        v3 = vals_v[pl.ds(3 * _LANES, _LANES)]
        d0 = v0 - fx
        d1 = v1 - fy
        d2 = v2 - bw
        d3 = v3 - bh
        acc_box = acc_box + (d0 * d0 + d1 * d1 + d2 * d2 + d3 * d3) * 0.25

        s_obj = _sigmoid((v0 + v1 + v2 + v3) * 0.25)

        def dup_step(k, dup):
            ck = plsc.load_gather(cells_v, [jnp.broadcast_to(k, (_LANES,))])
            hit = (my_cell == ck) & (k < my_rank)
            return dup | hit.astype(jnp.int32)

        dup = lax.fori_loop(0, _NOBJ, dup_step, jnp.zeros((_LANES,), jnp.int32))
        acc_obj = acc_obj + jnp.where(dup > 0, 0.0, 1.0 - 2.0 * s_obj) * inv_n

        def cls_step(s, acc):
            off = pl.multiple_of((s + 4) * _LANES, _LANES)
            v = vals_v[pl.ds(off, _LANES)]
            t = (cls_vec == s).astype(jnp.float32)
            bce = jnp.maximum(v, 0.0) - v * t + _softplus_neg(jnp.abs(v))
            return acc + bce * (1.0 / _NCLS)

        acc_cls = lax.fori_loop(0, _NCLS, cls_step, acc_cls)

    box_s = jnp.sum(acc_box)
    cls_s = jnp.sum(acc_cls)
    obj_s = jnp.sum(acc_obj)
    res = (jnp.where(iota == 0, box_s, 0.0)
           + jnp.where(iota == 1, cls_s, 0.0)
           + jnp.where(iota == 2, obj_s, 0.0))
    res_v[...] = res
    pltpu.sync_copy(res_v, out.at[pl.ds(pl.multiple_of(wid * _LANES, _LANES),
                                        _LANES)])


@jax.jit
def kernel(preds_0, preds_1, preds_2, targets_box, targets_cls):
    p0 = preds_0.reshape(-1)
    p1 = preds_1.reshape(-1)
    p2 = preds_2.reshape(-1)
    tb = jnp.transpose(targets_box, (2, 0, 1)).reshape(-1)
    tc = targets_cls.reshape(-1)

    sc_call = functools.partial(
        pl.kernel,
        out_type=jax.ShapeDtypeStruct((512,), jnp.float32),
        mesh=plsc.VectorSubcoreMesh(core_axis_name="c", subcore_axis_name="s"),
        compiler_params=pltpu.CompilerParams(needs_layout_passes=False),
        scratch_types=[
            pltpu.VMEM((3200,), jnp.float32),
            pltpu.VMEM((3200,), jnp.float32),
            pltpu.VMEM((3200,), jnp.float32),
            pltpu.VMEM((3200,), jnp.float32),
            pltpu.VMEM((1344,), jnp.int32),
            pltpu.VMEM((1344,), jnp.float32),
            pltpu.VMEM((32,), jnp.float32),
            pltpu.VMEM((32,), jnp.float32),
            pltpu.VMEM((16,), jnp.float32),
            pltpu.VMEM((16,), jnp.float32),
            pltpu.VMEM((16,), jnp.int32),
            pltpu.VMEM((32,), jnp.int32),
            pltpu.VMEM((16,), jnp.float32),
            pltpu.SemaphoreType.DMA,
        ],
    )(_sc_body)
    partials = sc_call(p0, p1, p2, tb, tc)
    p = jnp.sum(partials.reshape(32, 16), axis=0)
    return (7.5 * p[0] + 0.5 * p[1] + 1.0 * p[2]) * (1.0 / 3.0)


# speedup vs baseline: 1.6062x; 1.1588x over previous
import functools

import jax
import jax.numpy as jnp
from jax import lax
from jax.experimental import pallas as pl
from jax.experimental.pallas import tpu as pltpu
from jax.experimental.pallas import tpu_sc as plsc

_BS = 16
_NOBJ = 32
_NCLS = 80
_LANES = 16
_SEG = 1344

_LEVELS = ((80, 80), (40, 40), (20, 20))
_DOFF = (0, 3200, 4000)


def _sigmoid(x):
    return 1.0 / (1.0 + jnp.exp(-x))


def _softplus_neg(a):
    z = jnp.exp(-a)
    u = z / (z + 2.0)
    u2 = u * u
    return 2.0 * u * (1.0 + u2 * (1.0 / 3.0 + u2 * (1.0 / 5.0
                      + u2 * (1.0 / 7.0 + u2 * (1.0 / 9.0)))))


def _sc_body(pb0, pb1, pb2, pc0, pc1, pc2, tb, tcls, out,
             d0_v, d1_v, d2_v, d3_v, idx_v, vals_v,
             tbx_v, tby_v, tbw_v, tbh_v, cls_v, cells_v, res_v,
             sem_d0, sem_d1, sem_d2, sem_g0, sem_g1, sem_g2):
    wid = lax.axis_index("s") * 2 + lax.axis_index("c")
    b = wid // 2
    oh = wid % 2
    iota = lax.iota(jnp.int32, _LANES)
    boxes = (pb0, pb1, pb2)
    clss = (pc0, pc1, pc2)
    dbufs = (d0_v, d1_v, d2_v, d3_v)
    sems_d = (sem_d0, sem_d1, sem_d2)
    sems_g = (sem_g0, sem_g1, sem_g2)

    dense_waits = []
    for lvl, (h, w) in enumerate(_LEVELS):
        hw = h * w
        if lvl < 2:
            n_el = hw // 2
            el_off = oh * n_el
        else:
            n_el = hw
            el_off = 0
        for c in range(4):
            e0 = (b * 4 + c) * hw + el_off
            dense_waits.append(pltpu.async_copy(
                boxes[lvl].at[pl.ds(e0, n_el)],
                dbufs[c].at[pl.ds(_DOFF[lvl], n_el)], sems_d[lvl]))

    off_b = b * _NOBJ
    off_my = off_b + oh * _LANES
    pltpu.sync_copy(tb.at[pl.ds(off_b, _NOBJ)], tbx_v)
    pltpu.sync_copy(tb.at[pl.ds(512 + off_b, _NOBJ)], tby_v)
    pltpu.sync_copy(tb.at[pl.ds(1024 + off_my, _LANES)], tbw_v)
    pltpu.sync_copy(tb.at[pl.ds(1536 + off_my, _LANES)], tbh_v)
    pltpu.sync_copy(tcls.at[pl.ds(off_my, _LANES)], cls_v)

    x_lo = tbx_v[pl.ds(0, _LANES)]
    x_hi = tbx_v[pl.ds(_LANES, _LANES)]
    y_lo = tby_v[pl.ds(0, _LANES)]
    y_hi = tby_v[pl.ds(_LANES, _LANES)]
    bw = tbw_v[...]
    bh = tbh_v[...]
    cls_vec = cls_v[...]
    ohv = jnp.broadcast_to(oh, (_LANES,))
    is_lo = ohv == 0
    my_rank = iota + oh * _LANES

    fxs, fys, cells = [], [], []
    gather_waits = [[], [], []]
    for lvl, (h, w) in enumerate(_LEVELS):
        hw = h * w
        ls = lvl * _SEG
        cx_lo = x_lo * float(w)
        cy_lo = y_lo * float(h)
        gx_lo = jnp.clip(cx_lo.astype(jnp.int32), 0, w - 1)
        gy_lo = jnp.clip(cy_lo.astype(jnp.int32), 0, h - 1)
        cx_hi = x_hi * float(w)
        cy_hi = y_hi * float(h)
        gx_hi = jnp.clip(cx_hi.astype(jnp.int32), 0, w - 1)
        gy_hi = jnp.clip(cy_hi.astype(jnp.int32), 0, h - 1)
        cells_v[pl.ds(64 * lvl, _LANES)] = gy_lo * w + gx_lo
        cells_v[pl.ds(64 * lvl + _LANES, _LANES)] = gy_hi * w + gx_hi

        my_cx = jnp.where(is_lo, cx_lo, cx_hi)
        my_cy = jnp.where(is_lo, cy_lo, cy_hi)
        my_gx = jnp.where(is_lo, gx_lo, gx_hi)
        my_gy = jnp.where(is_lo, gy_lo, gy_hi)
        fxs.append(my_cx - my_gx.astype(jnp.float32))
        fys.append(my_cy - my_gy.astype(jnp.float32))
        sp = my_gy * w + my_gx
        cells.append(sp)

        for s in range(4):
            idx_v[pl.ds(ls + s * _LANES, _LANES)] = sp + ((b * 4 + s) * hw)
        cbase = (b * hw + sp) * _NCLS

        def cls_idx_step(j, carry, cbase=cbase, ls=ls):
            off = pl.multiple_of(ls + 64 + j * _LANES, _LANES)
            idx_v[pl.ds(off, _LANES)] = cbase + j
            return carry

        lax.fori_loop(0, _NCLS, cls_idx_step, 0)

        gather_waits[lvl].append(pltpu.async_copy(
            boxes[lvl].at[idx_v.at[pl.ds(ls, 64)]],
            vals_v.at[pl.ds(ls, 64)], sems_g[lvl]))
        for j in range(10):
            o = ls + 64 + j * 128
            gather_waits[lvl].append(pltpu.async_copy(
                clss[lvl].at[idx_v.at[pl.ds(o, 128)]],
                vals_v.at[pl.ds(o, 128)], sems_g[lvl]))

    acc_obj = jnp.zeros((_LANES,), jnp.float32)
    for lvl, (h, w) in enumerate(_LEVELS):
        hw = h * w
        inv_n = 1.0 / float(_BS * hw)
        scale = inv_n if lvl < 2 else inv_n * 0.5
        n_el = hw // 2 if lvl < 2 else hw
        doff = _DOFF[lvl]
        for _ in range(4):
            dense_waits[4 * lvl + _].wait()

        def dense_step(i, acc, doff=doff, scale=scale):
            base = pl.multiple_of(doff + i * _LANES, _LANES)
            v = (d0_v[pl.ds(base, _LANES)]
                 + d1_v[pl.ds(base, _LANES)]
                 + d2_v[pl.ds(base, _LANES)]
                 + d3_v[pl.ds(base, _LANES)]) * 0.25
            s = _sigmoid(v)
            return acc + s * s * scale

        acc_obj = lax.fori_loop(0, n_el // _LANES, dense_step, acc_obj)

    acc_box = jnp.zeros((_LANES,), jnp.float32)
    acc_cls = jnp.zeros((_LANES,), jnp.float32)
    for lvl, (h, w) in enumerate(_LEVELS):
        hw = h * w
        inv_n = 1.0 / float(_BS * hw)
        ls = lvl * _SEG
        for wgt in gather_waits[lvl]:
            wgt.wait()

        v0 = vals_v[pl.ds(ls, _LANES)]
        v1 = vals_v[pl.ds(ls + _LANES, _LANES)]
        v2 = vals_v[pl.ds(ls + 2 * _LANES, _LANES)]
        v3 = vals_v[pl.ds(ls + 3 * _LANES, _LANES)]
        d0 = v0 - fxs[lvl]
        d1 = v1 - fys[lvl]
        d2 = v2 - bw
        d3 = v3 - bh
        acc_box = acc_box + (d0 * d0 + d1 * d1 + d2 * d2 + d3 * d3) * 0.25

        s_obj = _sigmoid((v0 + v1 + v2 + v3) * 0.25)
        my_cell = cells[lvl]

        def dup_step(k, dup, lvl=lvl, my_cell=my_cell):
            ck = plsc.load_gather(
                cells_v, [jnp.broadcast_to(64 * lvl + k, (_LANES,))])
            hit = (my_cell == ck) & (k < my_rank)
            return dup | hit.astype(jnp.int32)

        dup = lax.fori_loop(0, _NOBJ, dup_step, jnp.zeros((_LANES,), jnp.int32))
        acc_obj = acc_obj + jnp.where(dup > 0, 0.0, 1.0 - 2.0 * s_obj) * inv_n

        def cls_step(s, acc, ls=ls):
            off = pl.multiple_of(ls + 64 + s * _LANES, _LANES)
            v = vals_v[pl.ds(off, _LANES)]
            t = (cls_vec == s).astype(jnp.float32)
            bce = jnp.maximum(v, 0.0) - v * t + _softplus_neg(jnp.abs(v))
            return acc + bce * (1.0 / _NCLS)

        acc_cls = lax.fori_loop(0, _NCLS, cls_step, acc_cls)

    box_s = jnp.sum(acc_box)
    cls_s = jnp.sum(acc_cls)
    obj_s = jnp.sum(acc_obj)
    res = (jnp.where(iota == 0, box_s, 0.0)
           + jnp.where(iota == 1, cls_s, 0.0)
           + jnp.where(iota == 2, obj_s, 0.0))
    res_v[...] = res
    pltpu.sync_copy(res_v, out.at[pl.ds(pl.multiple_of(wid * _LANES, _LANES),
                                        _LANES)])


@jax.jit
def kernel(preds_0, preds_1, preds_2, targets_box, targets_cls):
    pbs, pcs = [], []
    for p in (preds_0, preds_1, preds_2):
        pbs.append(p[:, :4].reshape(-1))
        pcs.append(jnp.transpose(p[:, 64:], (0, 2, 3, 1)).reshape(-1))
    tb = jnp.transpose(targets_box, (2, 0, 1)).reshape(-1)
    tc = targets_cls.reshape(-1)

    sc_call = functools.partial(
        pl.kernel,
        out_type=jax.ShapeDtypeStruct((512,), jnp.float32),
        mesh=plsc.VectorSubcoreMesh(core_axis_name="c", subcore_axis_name="s"),
        compiler_params=pltpu.CompilerParams(needs_layout_passes=False),
        scratch_types=[
            pltpu.VMEM((4400,), jnp.float32),
            pltpu.VMEM((4400,), jnp.float32),
            pltpu.VMEM((4400,), jnp.float32),
            pltpu.VMEM((4400,), jnp.float32),
            pltpu.VMEM((3 * _SEG,), jnp.int32),
            pltpu.VMEM((3 * _SEG,), jnp.float32),
            pltpu.VMEM((32,), jnp.float32),
            pltpu.VMEM((32,), jnp.float32),
            pltpu.VMEM((16,), jnp.float32),
            pltpu.VMEM((16,), jnp.float32),
            pltpu.VMEM((16,), jnp.int32),
            pltpu.VMEM((192,), jnp.int32),
            pltpu.VMEM((16,), jnp.float32),
            pltpu.SemaphoreType.DMA,
            pltpu.SemaphoreType.DMA,
            pltpu.SemaphoreType.DMA,
            pltpu.SemaphoreType.DMA,
            pltpu.SemaphoreType.DMA,
            pltpu.SemaphoreType.DMA,
        ],
    )(_sc_body)
    partials = sc_call(*pbs, *pcs, tb, tc)
    p = jnp.sum(partials.reshape(32, 16), axis=0)
    return (7.5 * p[0] + 0.5 * p[1] + 1.0 * p[2]) * (1.0 / 3.0)
